# baseline (speedup 1.0000x reference)
import jax
import jax.numpy as jnp
from jax import lax
from jax.experimental import pallas as pl
from jax.experimental.pallas import tpu as pltpu
from jax.experimental.pallas import tpu_sc as plsc

VOCAB = 100000
EMB = 32
NBLOCKS = 100
BATCH = 4096

C = 400
NBUF = 4


def _sc_body(hash_hbm, table_hbm, pos_hbm, out_hbm, idx_v, pos_v, pos_sh,
             buf_v, *sems):
    gsems = sems[:NBUF]
    osems = sems[NBUF:]
    info = plsc.get_sparse_core_info()
    nc = info.num_cores
    nw = nc * info.num_subcores
    per_w = (BATCH * NBLOCKS) // nw
    ngroup = per_w // C // NBUF

    sid = lax.axis_index("s")
    wid = sid * nc + lax.axis_index("c")
    base = wid * per_w

    pltpu.sync_copy(hash_hbm.at[wid], idx_v)

    @pl.when(sid == 0)
    def _():
        pltpu.sync_copy(pos_hbm, pos_v)
        for k in range(C // NBLOCKS):
            pltpu.sync_copy(pos_v, pos_sh.at[pl.ds(k * NBLOCKS, NBLOCKS)])

    plsc.subcore_barrier()

    def run_group(g, first):
        descs = []
        for b in range(NBUF):
            c = g * NBUF + b
            if not first:
                pltpu.make_async_copy(
                    buf_v.at[b], out_hbm.at[pl.ds(base, C)], osems[b]).wait()
            pltpu.sync_copy(pos_sh, buf_v.at[b])
            descs.append(pltpu.async_copy(
                table_hbm.at[idx_v.at[c]], buf_v.at[b],
                gsems[b], add=True))
        for b in range(NBUF):
            c = g * NBUF + b
            descs[b].wait()
            pltpu.async_copy(
                buf_v.at[b], out_hbm.at[pl.ds(base + c * C, C)], osems[b])

    run_group(0, True)

    def grp(g, carry):
        run_group(g, False)
        return carry

    lax.fori_loop(1, ngroup, grp, 0)

    for b in range(NBUF):
        pltpu.make_async_copy(
            buf_v.at[b], out_hbm.at[pl.ds(base, C)], osems[b]).wait()


def kernel(hash_ids, table, pos):
    pos2d = pos.reshape(NBLOCKS, EMB)
    mesh = plsc.VectorSubcoreMesh(core_axis_name="c", subcore_axis_name="s")
    info = plsc.get_sparse_core_info()
    nw = info.num_cores * info.num_subcores
    per_w = (BATCH * NBLOCKS) // nw
    ids3 = hash_ids.reshape(nw, per_w // C, C)

    run = pl.kernel(
        _sc_body,
        out_type=jax.ShapeDtypeStruct((BATCH * NBLOCKS, EMB), jnp.float32),
        mesh=mesh,
        scratch_types=[
            pltpu.VMEM((per_w // C, C), jnp.int32),
            pltpu.VMEM((NBLOCKS, EMB), jnp.float32),
            pltpu.VMEM_SHARED((C, EMB), jnp.float32),
            pltpu.VMEM((NBUF, C, EMB), jnp.float32),
        ] + [pltpu.SemaphoreType.DMA] * (2 * NBUF),
        compiler_params=pltpu.CompilerParams(use_tc_tiling_on_sc=False),
    )
    out = run(ids3, table, pos2d)
    return out.reshape(BATCH, NBLOCKS, EMB)


# speedup vs baseline: 1.3480x; 1.3480x over previous
import jax
import jax.numpy as jnp
from jax import lax
from jax.experimental import pallas as pl
from jax.experimental.pallas import tpu as pltpu
from jax.experimental.pallas import tpu_sc as plsc

VOCAB = 100000
EMB = 32
NBLOCKS = 100
BATCH = 4096
L = 16


def _sc_body(hash_hbm, table_hbm, pos_hbm, out_hbm, tbl_v, pos_v, idx_v,
             obuf_v, sem):
    info = plsc.get_sparse_core_info()
    nc = info.num_cores

    e = lax.axis_index("s") * nc + lax.axis_index("c")

    pltpu.sync_copy(table_hbm.at[e], tbl_v)
    pltpu.sync_copy(pos_hbm.at[e], pos_v)

    @pl.loop(0, NBLOCKS)
    def _n_loop(n):
        pltpu.sync_copy(hash_hbm.at[n], idx_v)
        posn = plsc.load_gather(pos_v, [jnp.full((L,), n, jnp.int32)])

        @pl.loop(0, BATCH // L, unroll=8)
        def _g_loop(g):
            idxs = idx_v[pl.ds(g * L, L)]
            vals = plsc.load_gather(tbl_v, [idxs])
            obuf_v[pl.ds(g * L, L)] = vals + posn

        pltpu.async_copy(obuf_v, out_hbm.at[n, e], sem).wait()


def kernel(hash_ids, table, pos):
    hash_t = hash_ids.T
    table_t = table.T
    pos_t = jnp.transpose(pos, (0, 2, 1)).reshape(EMB, NBLOCKS)
    mesh = plsc.VectorSubcoreMesh(core_axis_name="c", subcore_axis_name="s")

    run = pl.kernel(
        _sc_body,
        out_type=jax.ShapeDtypeStruct((NBLOCKS, EMB, BATCH), jnp.float32),
        mesh=mesh,
        scratch_types=[
            pltpu.VMEM((VOCAB,), jnp.float32),
            pltpu.VMEM((NBLOCKS,), jnp.float32),
            pltpu.VMEM((BATCH,), jnp.int32),
            pltpu.VMEM((BATCH,), jnp.float32),
            pltpu.SemaphoreType.DMA,
        ],
        compiler_params=pltpu.CompilerParams(needs_layout_passes=False),
    )
    out_t = run(hash_t, table_t, pos_t)
    return jnp.transpose(out_t, (2, 0, 1))


# speedup vs baseline: 4.3791x; 3.2485x over previous
import jax
import jax.numpy as jnp
from jax import lax
from jax.experimental import pallas as pl
from jax.experimental.pallas import tpu as pltpu
from jax.experimental.pallas import tpu_sc as plsc

VOCAB = 100000
EMB = 32
NBLOCKS = 100
BATCH = 4096
L = 16


def _sc_body(hash_hbm, table_hbm, pos_hbm, out_hbm, tbl_v, pos_v, idx_v,
             obuf_v, isem0, isem1, osem0, osem1):
    info = plsc.get_sparse_core_info()
    nc = info.num_cores
    isems = (isem0, isem1)
    osems = (osem0, osem1)

    e = lax.axis_index("s") * nc + lax.axis_index("c")

    pltpu.sync_copy(table_hbm.at[e], tbl_v)
    pltpu.sync_copy(pos_hbm.at[e], pos_v)

    pltpu.async_copy(hash_hbm.at[0], idx_v.at[0], isems[0])
    pltpu.async_copy(hash_hbm.at[1], idx_v.at[1], isems[1])

    def compute(k, slot):
        posn = plsc.load_gather(pos_v, [jnp.full((L,), k, jnp.int32)])

        @plsc.parallel_loop(0, BATCH // L, unroll=8)
        def _g_loop(g):
            idxs = idx_v[slot, pl.ds(g * L, L)]
            vals = plsc.load_gather(tbl_v, [idxs])
            obuf_v[slot, pl.ds(g * L, L)] = vals + posn

        pltpu.async_copy(obuf_v.at[slot], out_hbm.at[k, e], osems[slot])

    def half(n, k, slot):
        pltpu.make_async_copy(hash_hbm.at[k], idx_v.at[slot], isems[slot]).wait()

        @pl.when(n > 0)
        def _():
            pltpu.make_async_copy(
                obuf_v.at[slot], out_hbm.at[k, e], osems[slot]).wait()

        compute(k, slot)

        @pl.when(k + 2 < NBLOCKS)
        def _():
            pltpu.async_copy(hash_hbm.at[k + 2], idx_v.at[slot], isems[slot])

    @pl.loop(0, NBLOCKS, step=2)
    def _n_loop(n):
        half(n, n, 0)
        half(n, n + 1, 1)

    for slot in (0, 1):
        pltpu.make_async_copy(
            obuf_v.at[slot], out_hbm.at[0, e], osems[slot]).wait()


def kernel(hash_ids, table, pos):
    hash_t = hash_ids.T
    table_t = table.T
    pos_t = jnp.transpose(pos, (0, 2, 1)).reshape(EMB, NBLOCKS)
    mesh = plsc.VectorSubcoreMesh(core_axis_name="c", subcore_axis_name="s")

    run = pl.kernel(
        _sc_body,
        out_type=jax.ShapeDtypeStruct((NBLOCKS, EMB, BATCH), jnp.float32),
        mesh=mesh,
        scratch_types=[
            pltpu.VMEM((VOCAB,), jnp.float32),
            pltpu.VMEM((NBLOCKS,), jnp.float32),
            pltpu.VMEM((2, BATCH), jnp.int32),
            pltpu.VMEM((2, BATCH), jnp.float32),
            pltpu.SemaphoreType.DMA,
            pltpu.SemaphoreType.DMA,
            pltpu.SemaphoreType.DMA,
            pltpu.SemaphoreType.DMA,
        ],
        compiler_params=pltpu.CompilerParams(needs_layout_passes=False),
    )
    out_t = run(hash_t, table_t, pos_t)
    return jnp.transpose(out_t, (2, 0, 1))


# speedup vs baseline: 4.3844x; 1.0012x over previous
import jax
import jax.numpy as jnp
from jax import lax
from jax.experimental import pallas as pl
from jax.experimental.pallas import tpu as pltpu
from jax.experimental.pallas import tpu_sc as plsc

VOCAB = 100000
EMB = 32
NBLOCKS = 100
BATCH = 4096
L = 16


def _sc_body(hash_hbm, table_hbm, pos_hbm, out_hbm, tbl_v, pos_v, idx_v,
             obuf_v, isem0, isem1, osem0, osem1):
    info = plsc.get_sparse_core_info()
    nc = info.num_cores
    isems = (isem0, isem1)
    osems = (osem0, osem1)

    e = lax.axis_index("s") * nc + lax.axis_index("c")

    pltpu.sync_copy(table_hbm.at[e], tbl_v)
    pltpu.sync_copy(pos_hbm.at[e], pos_v)

    pltpu.async_copy(hash_hbm.at[0], idx_v.at[0], isems[0])
    pltpu.async_copy(hash_hbm.at[1], idx_v.at[1], isems[1])

    def compute(k, slot):
        posn = plsc.load_gather(pos_v, [jnp.full((L,), k, jnp.int32)])

        @plsc.parallel_loop(0, BATCH // L, unroll=16)
        def _g_loop(g):
            idxs = idx_v[slot, pl.ds(g * L, L)]
            vals = plsc.load_gather(tbl_v, [idxs])
            obuf_v[slot, pl.ds(g * L, L)] = vals + posn

        pltpu.async_copy(obuf_v.at[slot], out_hbm.at[k, e], osems[slot])

    def half(n, k, slot):
        pltpu.make_async_copy(hash_hbm.at[k], idx_v.at[slot], isems[slot]).wait()

        @pl.when(n > 0)
        def _():
            pltpu.make_async_copy(
                obuf_v.at[slot], out_hbm.at[k, e], osems[slot]).wait()

        compute(k, slot)

        @pl.when(k + 2 < NBLOCKS)
        def _():
            pltpu.async_copy(hash_hbm.at[k + 2], idx_v.at[slot], isems[slot])

    @pl.loop(0, NBLOCKS, step=2)
    def _n_loop(n):
        half(n, n, 0)
        half(n, n + 1, 1)

    for slot in (0, 1):
        pltpu.make_async_copy(
            obuf_v.at[slot], out_hbm.at[0, e], osems[slot]).wait()


def kernel(hash_ids, table, pos):
    hash_t = hash_ids.T
    table_t = table.T
    pos_t = jnp.transpose(pos, (0, 2, 1)).reshape(EMB, NBLOCKS)
    mesh = plsc.VectorSubcoreMesh(core_axis_name="c", subcore_axis_name="s")

    run = pl.kernel(
        _sc_body,
        out_type=jax.ShapeDtypeStruct((NBLOCKS, EMB, BATCH), jnp.float32),
        mesh=mesh,
        scratch_types=[
            pltpu.VMEM((VOCAB,), jnp.float32),
            pltpu.VMEM((NBLOCKS,), jnp.float32),
            pltpu.VMEM((2, BATCH), jnp.int32),
            pltpu.VMEM((2, BATCH), jnp.float32),
            pltpu.SemaphoreType.DMA,
            pltpu.SemaphoreType.DMA,
            pltpu.SemaphoreType.DMA,
            pltpu.SemaphoreType.DMA,
        ],
        compiler_params=pltpu.CompilerParams(needs_layout_passes=False),
    )
    out_t = run(hash_t, table_t, pos_t)
    return jnp.transpose(out_t, (2, 0, 1))
